# baseline (speedup 1.0000x reference)
import functools

import jax
import jax.numpy as jnp
from jax import lax
from jax.experimental import pallas as pl
from jax.experimental.pallas import tpu as pltpu
from jax.experimental.pallas import tpu_sc as plsc

_E = 512
_BS = 128
_N = 63
_LEAF0 = 31
_PADN = 64
_ROWS = _PADN * _BS

_NC, _NS = 2, 16
_NW = _NC * _NS
_BPW = _ROWS // _NW
_CH = 4
_CROWS = _BPW // _CH

@functools.cache
def _make_sc_gather():
    mesh = plsc.VectorSubcoreMesh(core_axis_name="c", subcore_axis_name="s")

    @functools.partial(
        pl.kernel,
        mesh=mesh,
        out_type=jax.ShapeDtypeStruct((_ROWS, _E), jnp.float32),
        scratch_types=[
            pltpu.VMEM((_CH, _CROWS), jnp.int32),
            pltpu.VMEM((_CROWS, _E), jnp.float32),
            pltpu.VMEM((_CROWS, _E), jnp.float32),
            pltpu.SemaphoreType.DMA,
            pltpu.SemaphoreType.DMA,
        ],
    )
    def _sc_gather(emb_hbm, idx_hbm, out_hbm, idx_v, buf0, buf1, sem0, sem1):
        wid = lax.axis_index("s") * _NC + lax.axis_index("c")
        base = wid * _BPW
        pltpu.sync_copy(idx_hbm.at[pl.ds(wid * _CH, _CH)], idx_v)
        bufs = (buf0, buf1)
        sems = (sem0, sem1)
        copies = [None, None]
        copies[0] = pltpu.async_copy(emb_hbm.at[idx_v.at[0]], buf0, sem0)
        for c in range(_CH):
            p = c & 1
            if c + 1 < _CH:
                q = (c + 1) & 1
                copies[q] = pltpu.async_copy(
                    emb_hbm.at[idx_v.at[c + 1]], bufs[q], sems[q])
            copies[p].wait()
            pltpu.sync_copy(bufs[p],
                            out_hbm.at[pl.ds(base + c * _CROWS, _CROWS)])

    return _sc_gather


def _tree_body(x_ref, wih_t_ref, whh_t_ref, bih_ref, bhh_ref, sw_ref, sb_ref,
               cw_ref, out_ref, h_all, macc):
    g = pl.program_id(0)
    node = _N - 1 - g
    row = node * _BS
    x = x_ref[pl.ds(row, _BS), :].astype(jnp.bfloat16)
    gi = jnp.dot(x, wih_t_ref[:, :].astype(jnp.bfloat16),
                 preferred_element_type=jnp.float32)
    gi = gi + bih_ref[:, :]
    i_r = gi[:, :_E]
    i_z = gi[:, _E:2 * _E]
    i_n = gi[:, 2 * _E:]

    @pl.when(node >= _LEAF0)
    def _leaf():
        r = jax.nn.sigmoid(i_r + bhh_ref[:, :_E])
        z = jax.nn.sigmoid(i_z + bhh_ref[:, _E:2 * _E])
        n = jnp.tanh(i_n + r * bhh_ref[:, 2 * _E:])
        h_all[pl.ds(row, _BS), :] = (1.0 - z) * n

    @pl.when(node < _LEAF0)
    def _internal():
        c1 = 2 * node + 1
        h1 = h_all[pl.ds(c1 * _BS, _BS), :]
        h2 = h_all[pl.ds((c1 + 1) * _BS, _BS), :]
        sw_b = sw_ref[:, :].astype(jnp.bfloat16)
        u1 = jnp.tanh(jnp.dot(h1.astype(jnp.bfloat16), sw_b,
                              preferred_element_type=jnp.float32) + sb_ref[:, :])
        u2 = jnp.tanh(jnp.dot(h2.astype(jnp.bfloat16), sw_b,
                              preferred_element_type=jnp.float32) + sb_ref[:, :])
        s1 = jnp.tanh(jnp.sum(u1 * cw_ref[:, :], axis=1, keepdims=True))
        s2 = jnp.tanh(jnp.sum(u2 * cw_ref[:, :], axis=1, keepdims=True))
        w1 = jax.nn.sigmoid(s1 - s2)
        h0 = w1 * h1 + (1.0 - w1) * h2
        gh = jnp.dot(h0.astype(jnp.bfloat16),
                     whh_t_ref[:, :].astype(jnp.bfloat16),
                     preferred_element_type=jnp.float32)
        gh = gh + bhh_ref[:, :]
        r = jax.nn.sigmoid(i_r + gh[:, :_E])
        z = jax.nn.sigmoid(i_z + gh[:, _E:2 * _E])
        n = jnp.tanh(i_n + r * gh[:, 2 * _E:])
        h_all[pl.ds(row, _BS), :] = (1.0 - z) * n + z * h0

    h = h_all[pl.ds(row, _BS), :]

    @pl.when(g == 0)
    def _init():
        macc[:, :] = h

    @pl.when(g > 0)
    def _acc():
        macc[:, :] = jnp.maximum(macc[:, :], h)

    @pl.when(g == _N - 1)
    def _fin():
        out_ref[:, :] = macc[:, :]


def _tree_gru(x_pad, wih_t, whh_t, bih_r, bhh_r, sw, sb, cw_r):
    return pl.pallas_call(
        _tree_body,
        grid=(_N,),
        in_specs=[pl.BlockSpec(memory_space=pltpu.VMEM)] * 8,
        out_specs=pl.BlockSpec(memory_space=pltpu.VMEM),
        out_shape=jax.ShapeDtypeStruct((_BS, _E), jnp.float32),
        scratch_shapes=[
            pltpu.VMEM((_N * _BS, _E), jnp.float32),
            pltpu.VMEM((_BS, _E), jnp.float32),
        ],
        compiler_params=pltpu.CompilerParams(
            dimension_semantics=("arbitrary",)),
    )(x_pad, wih_t, whh_t, bih_r, bhh_r, sw, sb, cw_r)


def kernel(node_ids, emb, Wih, Whh, bih, bhh, sent_w, sent_b, ctx_w):
    ids = node_ids.reshape(-1).astype(jnp.int32)
    ids_pad = jnp.concatenate(
        [ids, jnp.zeros((_ROWS - _N * _BS,), jnp.int32)])
    idx2 = ids_pad.reshape(_NW * _CH, _CROWS)
    x_pad = _make_sc_gather()(emb, idx2)
    return _tree_gru(
        x_pad, Wih.T, Whh.T,
        bih.reshape(1, 3 * _E), bhh.reshape(1, 3 * _E),
        sent_w, sent_b, ctx_w.reshape(1, _E))


# speedup vs baseline: 1.4139x; 1.4139x over previous
import functools

import jax
import jax.numpy as jnp
from jax import lax
from jax.experimental import pallas as pl
from jax.experimental.pallas import tpu as pltpu
from jax.experimental.pallas import tpu_sc as plsc

_E = 512
_BS = 128
_N = 63
_PADN = 64
_ROWS = _PADN * _BS

_NC, _NS = 2, 16
_NW = _NC * _NS
_BPW = _ROWS // _NW
_CH = 4
_CROWS = _BPW // _CH


@functools.cache
def _make_sc_gather():
    mesh = plsc.VectorSubcoreMesh(core_axis_name="c", subcore_axis_name="s")

    @functools.partial(
        pl.kernel,
        mesh=mesh,
        out_type=jax.ShapeDtypeStruct((_ROWS, _E), jnp.float32),
        scratch_types=[
            pltpu.VMEM((_CH, _CROWS), jnp.int32),
            pltpu.VMEM((_CROWS, _E), jnp.float32),
            pltpu.VMEM((_CROWS, _E), jnp.float32),
            pltpu.SemaphoreType.DMA,
            pltpu.SemaphoreType.DMA,
        ],
    )
    def _sc_gather(emb_hbm, idx_hbm, out_hbm, idx_v, buf0, buf1, sem0, sem1):
        wid = lax.axis_index("s") * _NC + lax.axis_index("c")
        base = wid * _BPW
        pltpu.sync_copy(idx_hbm.at[pl.ds(wid * _CH, _CH)], idx_v)
        bufs = (buf0, buf1)
        sems = (sem0, sem1)
        copies = [None, None]
        copies[0] = pltpu.async_copy(emb_hbm.at[idx_v.at[0]], buf0, sem0)
        for c in range(_CH):
            p = c & 1
            if c + 1 < _CH:
                q = (c + 1) & 1
                copies[q] = pltpu.async_copy(
                    emb_hbm.at[idx_v.at[c + 1]], bufs[q], sems[q])
            copies[p].wait()
            pltpu.sync_copy(bufs[p],
                            out_hbm.at[pl.ds(base + c * _CROWS, _CROWS)])

    return _sc_gather


def _gru_combine(gi, gh, h0):
    r = jax.nn.sigmoid(gi[:, :_E] + gh[:, :_E])
    z = jax.nn.sigmoid(gi[:, _E:2 * _E] + gh[:, _E:2 * _E])
    n = jnp.tanh(gi[:, 2 * _E:] + r * gh[:, 2 * _E:])
    return (1.0 - z) * n + z * h0 if h0 is not None else (1.0 - z) * n


def _tree_body(x_ref, wih_t_ref, whh_t_ref, bih_ref, bhh_ref, sw_ref, sb_ref,
               cw_ref, out_ref, h_all, macc):
    g = pl.program_id(0)

    def gi_at(prow, m):
        x = x_ref[pl.ds(prow, m), :]
        return jnp.dot(x, wih_t_ref[:, :],
                       preferred_element_type=jnp.float32) + bih_ref[:, :]

    def fold_max(h, nn):
        m = jnp.max(h.reshape(nn, _BS, _E), axis=0) if nn > 1 else h

        @pl.when(g == 0)
        def _init():
            macc[:, :] = m

        @pl.when(g > 0)
        def _acc():
            macc[:, :] = jnp.maximum(macc[:, :], m)

    def leaf_step(prow, nn):
        m = nn * _BS
        gi = gi_at(prow, m)
        gh = jnp.broadcast_to(bhh_ref[:, :], (m, 3 * _E))
        h = _gru_combine(gi, gh, None)
        h_all[pl.ds(prow, m), :] = h
        fold_max(h, nn)

    def internal_step(prow, crow, nn):
        m = nn * _BS
        hc = h_all[pl.ds(crow, 2 * m), :]
        u = jnp.tanh(jnp.dot(hc, sw_ref[:, :],
                             preferred_element_type=jnp.float32)
                     + sb_ref[:, :])
        s = jnp.tanh(jnp.sum(u * cw_ref[:, :], axis=1, keepdims=True))
        s4 = s.reshape(nn, 2, _BS, 1)
        w1 = jax.nn.sigmoid(s4[:, 0] - s4[:, 1])
        hc4 = hc.reshape(nn, 2, _BS, _E)
        h0 = (w1 * hc4[:, 0] + (1.0 - w1) * hc4[:, 1]).reshape(m, _E)
        gi = gi_at(prow, m)
        gh = jnp.dot(h0, whh_t_ref[:, :],
                     preferred_element_type=jnp.float32) + bhh_ref[:, :]
        h = _gru_combine(gi, gh, h0)
        h_all[pl.ds(prow, m), :] = h
        fold_max(h, nn)

    @pl.when(g < 4)
    def _leaves():
        leaf_step(3968 + g * 1024, 8)

    @pl.when((g == 4) | (g == 5))
    def _lvl4():
        half = g - 4
        internal_step(1920 + half * 1024, 3968 + half * 2048, 8)

    @pl.when(g == 6)
    def _lvl3():
        internal_step(896, 1920, 8)

    @pl.when(g == 7)
    def _lvl2():
        internal_step(384, 896, 4)

    @pl.when(g == 8)
    def _lvl1():
        internal_step(128, 384, 2)

    @pl.when(g == 9)
    def _lvl0():
        internal_step(0, 128, 1)
        out_ref[:, :] = macc[:, :]


def _tree_gru(x_pad, wih_t, whh_t, bih_r, bhh_r, sw, sb, cw_r):
    return pl.pallas_call(
        _tree_body,
        grid=(10,),
        in_specs=[pl.BlockSpec(memory_space=pltpu.VMEM)] * 8,
        out_specs=pl.BlockSpec(memory_space=pltpu.VMEM),
        out_shape=jax.ShapeDtypeStruct((_BS, _E), jnp.float32),
        scratch_shapes=[
            pltpu.VMEM((_N * _BS, _E), jnp.float32),
            pltpu.VMEM((_BS, _E), jnp.float32),
        ],
        compiler_params=pltpu.CompilerParams(
            dimension_semantics=("arbitrary",)),
    )(x_pad, wih_t, whh_t, bih_r, bhh_r, sw, sb, cw_r)


def kernel(node_ids, emb, Wih, Whh, bih, bhh, sent_w, sent_b, ctx_w):
    ids = node_ids.reshape(-1).astype(jnp.int32)
    ids_pad = jnp.concatenate(
        [ids, jnp.zeros((_ROWS - _N * _BS,), jnp.int32)])
    idx2 = ids_pad.reshape(_NW * _CH, _CROWS)
    x_pad = _make_sc_gather()(emb, idx2)
    return _tree_gru(
        x_pad, Wih.T, Whh.T,
        bih.reshape(1, 3 * _E), bhh.reshape(1, 3 * _E),
        sent_w, sent_b, ctx_w.reshape(1, _E))


# speedup vs baseline: 1.4419x; 1.0198x over previous
import functools

import jax
import jax.numpy as jnp
from jax import lax
from jax.experimental import pallas as pl
from jax.experimental.pallas import tpu as pltpu
from jax.experimental.pallas import tpu_sc as plsc

_E = 512
_BS = 128
_N = 63
_PADN = 64
_ROWS = _PADN * _BS

_NC, _NS = 2, 16
_NW = _NC * _NS
_BPW = _ROWS // _NW
_CH = 4
_CROWS = _BPW // _CH


@functools.cache
def _make_sc_gather():
    mesh = plsc.VectorSubcoreMesh(core_axis_name="c", subcore_axis_name="s")

    @functools.partial(
        pl.kernel,
        mesh=mesh,
        out_type=jax.ShapeDtypeStruct((_ROWS, _E), jnp.float32),
        scratch_types=[
            pltpu.VMEM((_CH, _CROWS), jnp.int32),
            pltpu.VMEM((_CROWS, _E), jnp.float32),
            pltpu.VMEM((_CROWS, _E), jnp.float32),
            pltpu.SemaphoreType.DMA,
            pltpu.SemaphoreType.DMA,
        ],
    )
    def _sc_gather(emb_hbm, idx_hbm, out_hbm, idx_v, buf0, buf1, sem0, sem1):
        wid = lax.axis_index("s") * _NC + lax.axis_index("c")
        base = wid * _BPW
        pltpu.sync_copy(idx_hbm.at[pl.ds(wid * _CH, _CH)], idx_v)
        bufs = (buf0, buf1)
        sems = (sem0, sem1)
        copies = [None, None]
        copies[0] = pltpu.async_copy(emb_hbm.at[idx_v.at[0]], buf0, sem0)
        for c in range(_CH):
            p = c & 1
            if c + 1 < _CH:
                q = (c + 1) & 1
                copies[q] = pltpu.async_copy(
                    emb_hbm.at[idx_v.at[c + 1]], bufs[q], sems[q])
            copies[p].wait()
            pltpu.sync_copy(bufs[p],
                            out_hbm.at[pl.ds(base + c * _CROWS, _CROWS)])

    return _sc_gather


def _dot_t(a, b_t):
    return lax.dot_general(a, b_t, (((1,), (1,)), ((), ())),
                           preferred_element_type=jnp.float32)


def _dot(a, b):
    return lax.dot_general(a, b, (((1,), (0,)), ((), ())),
                           preferred_element_type=jnp.float32)


def _gru_combine(gi, gh, h0):
    r = jax.nn.sigmoid(gi[:, :_E] + gh[:, :_E])
    z = jax.nn.sigmoid(gi[:, _E:2 * _E] + gh[:, _E:2 * _E])
    n = jnp.tanh(gi[:, 2 * _E:] + r * gh[:, 2 * _E:])
    return (1.0 - z) * n + z * h0 if h0 is not None else (1.0 - z) * n


def _tree_body(x_ref, wih_ref, whh_ref, bih_ref, bhh_ref, sw_ref, sb_ref,
               cw_ref, out_ref, h_all, macc):
    g = pl.program_id(0)

    def gi_at(prow, m):
        x = x_ref[pl.ds(prow, m), :]
        return _dot_t(x, wih_ref[:, :]) + bih_ref[:, :]

    def fold_max(h, nn):
        m = jnp.max(h.reshape(nn, _BS, _E), axis=0) if nn > 1 else h

        @pl.when(g == 0)
        def _init():
            macc[:, :] = m

        @pl.when(g > 0)
        def _acc():
            macc[:, :] = jnp.maximum(macc[:, :], m)

    def leaf_step(prow, nn):
        m = nn * _BS
        gi = gi_at(prow, m)
        gh = jnp.broadcast_to(bhh_ref[:, :], (m, 3 * _E))
        h = _gru_combine(gi, gh, None)
        h_all[pl.ds(prow, m), :] = h
        fold_max(h, nn)

    def internal_step(prow, crow, nn):
        m = nn * _BS
        hc = h_all[pl.ds(crow, 2 * m), :]
        u = jnp.tanh(_dot(hc, sw_ref[:, :]) + sb_ref[:, :])
        s = jnp.tanh(jnp.sum(u * cw_ref[:, :], axis=1, keepdims=True))
        s4 = s.reshape(nn, 2, _BS, 1)
        w1 = jax.nn.sigmoid(s4[:, 0] - s4[:, 1])
        hc4 = hc.reshape(nn, 2, _BS, _E)
        h0 = (w1 * hc4[:, 0] + (1.0 - w1) * hc4[:, 1]).reshape(m, _E)
        gi = gi_at(prow, m)
        gh = _dot_t(h0, whh_ref[:, :]) + bhh_ref[:, :]
        h = _gru_combine(gi, gh, h0)
        h_all[pl.ds(prow, m), :] = h
        fold_max(h, nn)

    @pl.when(g < 4)
    def _leaves():
        leaf_step(3968 + g * 1024, 8)

    @pl.when((g == 4) | (g == 5))
    def _lvl4():
        half = g - 4
        internal_step(1920 + half * 1024, 3968 + half * 2048, 8)

    @pl.when(g == 6)
    def _lvl3():
        internal_step(896, 1920, 8)

    @pl.when(g == 7)
    def _lvl2():
        internal_step(384, 896, 4)

    @pl.when(g == 8)
    def _lvl1():
        internal_step(128, 384, 2)

    @pl.when(g == 9)
    def _lvl0():
        internal_step(0, 128, 1)
        out_ref[:, :] = macc[:, :]


def _tree_gru(x_pad, wih, whh, bih_r, bhh_r, sw, sb, cw_r):
    return pl.pallas_call(
        _tree_body,
        grid=(10,),
        in_specs=[pl.BlockSpec(memory_space=pltpu.VMEM)] * 8,
        out_specs=pl.BlockSpec(memory_space=pltpu.VMEM),
        out_shape=jax.ShapeDtypeStruct((_BS, _E), jnp.float32),
        scratch_shapes=[
            pltpu.VMEM((_N * _BS, _E), jnp.float32),
            pltpu.VMEM((_BS, _E), jnp.float32),
        ],
        compiler_params=pltpu.CompilerParams(
            dimension_semantics=("arbitrary",)),
    )(x_pad, wih, whh, bih_r, bhh_r, sw, sb, cw_r)


def kernel(node_ids, emb, Wih, Whh, bih, bhh, sent_w, sent_b, ctx_w):
    ids = node_ids.reshape(-1).astype(jnp.int32)
    ids_pad = jnp.concatenate(
        [ids, jnp.zeros((_ROWS - _N * _BS,), jnp.int32)])
    idx2 = ids_pad.reshape(_NW * _CH, _CROWS)
    x_pad = _make_sc_gather()(emb, idx2)
    return _tree_gru(
        x_pad, Wih, Whh,
        bih.reshape(1, 3 * _E), bhh.reshape(1, 3 * _E),
        sent_w, sent_b, ctx_w.reshape(1, _E))


# speedup vs baseline: 1.4791x; 1.0258x over previous
import functools

import jax
import jax.numpy as jnp
from jax import lax
from jax.experimental import pallas as pl
from jax.experimental.pallas import tpu as pltpu
from jax.experimental.pallas import tpu_sc as plsc

_E = 512
_BS = 128
_N = 63
_PADN = 64
_ROWS = _PADN * _BS

_NC, _NS = 2, 16
_NW = _NC * _NS
_BPW = _ROWS // _NW
_CH = 4
_CROWS = _BPW // _CH


@functools.cache
def _make_sc_gather():
    mesh = plsc.VectorSubcoreMesh(core_axis_name="c", subcore_axis_name="s")

    @functools.partial(
        pl.kernel,
        mesh=mesh,
        out_type=jax.ShapeDtypeStruct((_ROWS, _E), jnp.float32),
        scratch_types=[
            pltpu.VMEM((_CH, _CROWS), jnp.int32),
            pltpu.VMEM((_CROWS, _E), jnp.float32),
            pltpu.VMEM((_CROWS, _E), jnp.float32),
            pltpu.SemaphoreType.DMA,
            pltpu.SemaphoreType.DMA,
        ],
    )
    def _sc_gather(emb_hbm, idx_hbm, out_hbm, idx_v, buf0, buf1, sem0, sem1):
        wid = lax.axis_index("s") * _NC + lax.axis_index("c")
        base = wid * _BPW
        pltpu.sync_copy(idx_hbm.at[pl.ds(wid * _CH, _CH)], idx_v)
        bufs = (buf0, buf1)
        sems = (sem0, sem1)
        copies = [None, None]
        copies[0] = pltpu.async_copy(emb_hbm.at[idx_v.at[0]], buf0, sem0)
        for c in range(_CH):
            p = c & 1
            if c + 1 < _CH:
                q = (c + 1) & 1
                copies[q] = pltpu.async_copy(
                    emb_hbm.at[idx_v.at[c + 1]], bufs[q], sems[q])
            copies[p].wait()
            pltpu.sync_copy(bufs[p],
                            out_hbm.at[pl.ds(base + c * _CROWS, _CROWS)])

    return _sc_gather


def _dot_t(a, b_t):
    return lax.dot_general(a, b_t, (((1,), (1,)), ((), ())),
                           preferred_element_type=jnp.float32)


def _dot(a, b):
    return lax.dot_general(a, b, (((1,), (0,)), ((), ())),
                           preferred_element_type=jnp.float32)


def _gru_combine(gi, gh, h0):
    r = jax.nn.sigmoid(gi[:, :_E] + gh[:, :_E])
    z = jax.nn.sigmoid(gi[:, _E:2 * _E] + gh[:, _E:2 * _E])
    n = jnp.tanh(gi[:, 2 * _E:] + r * gh[:, 2 * _E:])
    return (1.0 - z) * n + z * h0 if h0 is not None else (1.0 - z) * n


def _tree_body(x_ref, wih_ref, whh_ref, bih_ref, bhh_ref, sw_ref, sb_ref,
               cw_ref, out_ref, h_all):
    bih = bih_ref[:]
    bhh = bhh_ref[:]

    def gi_at(prow, m):
        x = x_ref[pl.ds(prow, m), :]
        return _dot_t(x, wih_ref[:, :]) + bih

    def leaf_step(prow, nn):
        m = nn * _BS
        gi = gi_at(prow, m)
        gh = jnp.broadcast_to(bhh, (m, 3 * _E))
        h = _gru_combine(gi, gh, None)
        h_all[pl.ds(prow, m), :] = h
        return h

    def internal_step(prow, crow, nn):
        m = nn * _BS
        hc = h_all[pl.ds(crow, 2 * m), :]
        u = jnp.tanh(_dot(hc, sw_ref[:, :]) + sb_ref[:, :])
        s = jnp.tanh(_dot(u, cw_ref[:, :]))
        s4 = s.reshape(nn, 2, _BS, 1)
        w1 = jax.nn.sigmoid(s4[:, 0] - s4[:, 1])
        hc4 = hc.reshape(nn, 2, _BS, _E)
        h0 = (w1 * hc4[:, 0] + (1.0 - w1) * hc4[:, 1]).reshape(m, _E)
        gi = gi_at(prow, m)
        gh = _dot_t(h0, whh_ref[:, :]) + bhh
        h = _gru_combine(gi, gh, h0)
        h_all[pl.ds(prow, m), :] = h
        return h

    def fold(acc, h, nn):
        m = jnp.max(h.reshape(nn, _BS, _E), axis=0) if nn > 1 else h
        return m if acc is None else jnp.maximum(acc, m)

    acc = None
    for k in range(4):
        acc = fold(acc, leaf_step(3968 + k * 1024, 8), 8)
    for half in range(2):
        acc = fold(acc, internal_step(1920 + half * 1024,
                                      3968 + half * 2048, 8), 8)
    acc = fold(acc, internal_step(896, 1920, 8), 8)
    acc = fold(acc, internal_step(384, 896, 4), 4)
    acc = fold(acc, internal_step(128, 384, 2), 2)
    acc = fold(acc, internal_step(0, 128, 1), 1)
    out_ref[:, :] = acc


def _tree_gru(x_pad, wih, whh, bih, bhh, sw, sb, cw):
    return pl.pallas_call(
        _tree_body,
        in_specs=[pl.BlockSpec(memory_space=pltpu.VMEM)] * 8,
        out_specs=pl.BlockSpec(memory_space=pltpu.VMEM),
        out_shape=jax.ShapeDtypeStruct((_BS, _E), jnp.float32),
        scratch_shapes=[
            pltpu.VMEM((_N * _BS, _E), jnp.float32),
        ],
    )(x_pad, wih, whh, bih, bhh, sw, sb, cw)


def kernel(node_ids, emb, Wih, Whh, bih, bhh, sent_w, sent_b, ctx_w):
    ids = node_ids.reshape(-1).astype(jnp.int32)
    ids_pad = jnp.concatenate(
        [ids, jnp.zeros((_ROWS - _N * _BS,), jnp.int32)])
    idx2 = ids_pad.reshape(_NW * _CH, _CROWS)
    x_pad = _make_sc_gather()(emb, idx2)
    return _tree_gru(x_pad, Wih, Whh, bih, bhh, sent_w, sent_b, ctx_w)


# speedup vs baseline: 1.4864x; 1.0049x over previous
import functools

import jax
import jax.numpy as jnp
from jax import lax
from jax.experimental import pallas as pl
from jax.experimental.pallas import tpu as pltpu
from jax.experimental.pallas import tpu_sc as plsc

_E = 512
_BS = 128
_N = 63
_PADN = 64
_ROWS = _PADN * _BS

_NC, _NS = 2, 16
_NW = _NC * _NS
_BPW = _ROWS // _NW
_CH = 4
_CROWS = _BPW // _CH


@functools.cache
def _make_sc_gather():
    mesh = plsc.VectorSubcoreMesh(core_axis_name="c", subcore_axis_name="s")

    @functools.partial(
        pl.kernel,
        mesh=mesh,
        out_type=jax.ShapeDtypeStruct((_ROWS, _E), jnp.float32),
        scratch_types=[
            pltpu.VMEM((_CH, _CROWS), jnp.int32),
            pltpu.VMEM((_CROWS, _E), jnp.float32),
            pltpu.VMEM((_CROWS, _E), jnp.float32),
            pltpu.SemaphoreType.DMA,
            pltpu.SemaphoreType.DMA,
        ],
    )
    def _sc_gather(emb_hbm, idx_hbm, out_hbm, idx_v, buf0, buf1, sem0, sem1):
        wid = lax.axis_index("s") * _NC + lax.axis_index("c")
        base = wid * _BPW
        pltpu.sync_copy(idx_hbm.at[pl.ds(wid * _CH, _CH)], idx_v)
        bufs = (buf0, buf1)
        sems = (sem0, sem1)
        copies = [None, None]
        copies[0] = pltpu.async_copy(emb_hbm.at[idx_v.at[0]], buf0, sem0)
        for c in range(_CH):
            p = c & 1
            if c + 1 < _CH:
                q = (c + 1) & 1
                copies[q] = pltpu.async_copy(
                    emb_hbm.at[idx_v.at[c + 1]], bufs[q], sems[q])
            copies[p].wait()
            pltpu.sync_copy(bufs[p],
                            out_hbm.at[pl.ds(base + c * _CROWS, _CROWS)])

    return _sc_gather


def _dot_t(a, b_t):
    return lax.dot_general(a, b_t, (((1,), (1,)), ((), ())),
                           preferred_element_type=jnp.float32)


def _dot(a, b):
    return lax.dot_general(a, b, (((1,), (0,)), ((), ())),
                           preferred_element_type=jnp.float32)


def _gru_combine(gi, gh, h0):
    r = jax.nn.sigmoid(gi[:, :_E] + gh[:, :_E])
    z = jax.nn.sigmoid(gi[:, _E:2 * _E] + gh[:, _E:2 * _E])
    n = jnp.tanh(gi[:, 2 * _E:] + r * gh[:, 2 * _E:])
    return (1.0 - z) * n + z * h0 if h0 is not None else (1.0 - z) * n


def _tree_body(x_ref, wih_ref, whh_ref, bih_ref, bhh_ref, sw_ref, sb_ref,
               cw_ref, out_ref, h_all):
    bih = bih_ref[:]
    bhh = bhh_ref[:]
    wih_b = wih_ref[:, :]
    whh_b = whh_ref[:, :]
    sw_b = sw_ref[:, :]
    cw_b = cw_ref[:, :]

    def gi_at(prow, m):
        x = x_ref[pl.ds(prow, m), :]
        return _dot_t(x, wih_b) + bih

    def leaf_step(prow, nn):
        m = nn * _BS
        gi = gi_at(prow, m)
        gh = jnp.broadcast_to(bhh, (m, 3 * _E))
        h = _gru_combine(gi, gh, None)
        h_all[pl.ds(prow, m), :] = h
        return h

    def internal_step(prow, crow, nn):
        m = nn * _BS
        hc = h_all[pl.ds(crow, 2 * m), :]
        u = jnp.tanh(_dot(hc, sw_b) + sb_ref[:, :])
        s = jnp.tanh(_dot(u, cw_b))
        s4 = s.reshape(nn, 2, _BS, 1)
        w1 = jax.nn.sigmoid(s4[:, 0] - s4[:, 1])
        hc4 = hc.reshape(nn, 2, _BS, _E)
        h0 = (w1 * hc4[:, 0] + (1.0 - w1) * hc4[:, 1]).reshape(m, _E)
        gi = gi_at(prow, m)
        gh = _dot_t(h0, whh_b) + bhh
        h = _gru_combine(gi, gh, h0)
        h_all[pl.ds(prow, m), :] = h
        return h

    def fold(acc, h, nn):
        m = jnp.max(h.reshape(nn, _BS, _E), axis=0) if nn > 1 else h
        return m if acc is None else jnp.maximum(acc, m)

    acc = None
    for k in range(4):
        acc = fold(acc, leaf_step(3968 + k * 1024, 8), 8)
    for half in range(2):
        acc = fold(acc, internal_step(1920 + half * 1024,
                                      3968 + half * 2048, 8), 8)
    acc = fold(acc, internal_step(896, 1920, 8), 8)
    acc = fold(acc, internal_step(384, 896, 4), 4)
    acc = fold(acc, internal_step(128, 384, 2), 2)
    acc = fold(acc, internal_step(0, 128, 1), 1)
    out_ref[:, :] = acc


def _tree_gru(x_pad, wih, whh, bih, bhh, sw, sb, cw):
    return pl.pallas_call(
        _tree_body,
        in_specs=[pl.BlockSpec(memory_space=pltpu.VMEM)] * 8,
        out_specs=pl.BlockSpec(memory_space=pltpu.VMEM),
        out_shape=jax.ShapeDtypeStruct((_BS, _E), jnp.float32),
        scratch_shapes=[
            pltpu.VMEM((_N * _BS, _E), jnp.float32),
        ],
    )(x_pad, wih, whh, bih, bhh, sw, sb, cw)


def kernel(node_ids, emb, Wih, Whh, bih, bhh, sent_w, sent_b, ctx_w):
    ids = node_ids.reshape(-1).astype(jnp.int32)
    ids_pad = jnp.concatenate(
        [ids, jnp.zeros((_ROWS - _N * _BS,), jnp.int32)])
    idx2 = ids_pad.reshape(_NW * _CH, _CROWS)
    x_pad = _make_sc_gather()(emb, idx2)
    return _tree_gru(x_pad, Wih, Whh, bih, bhh, sent_w, sent_b, ctx_w)


# speedup vs baseline: 1.6955x; 1.1407x over previous
import functools

import jax
import jax.numpy as jnp
from jax import lax
from jax.experimental import pallas as pl
from jax.experimental.pallas import tpu as pltpu
from jax.experimental.pallas import tpu_sc as plsc

_E = 512
_BS = 128
_N = 63
_PADN = 64
_ROWS = _PADN * _BS

_NC, _NS = 2, 16
_NW = _NC * _NS
_BPW = _ROWS // _NW
_CH = 4
_CROWS = _BPW // _CH


@functools.cache
def _make_sc_gather():
    mesh = plsc.VectorSubcoreMesh(core_axis_name="c", subcore_axis_name="s")

    @functools.partial(
        pl.kernel,
        mesh=mesh,
        out_type=jax.ShapeDtypeStruct((_N * _BS, _E), jnp.float32),
        scratch_types=[
            pltpu.VMEM((_BPW,), jnp.int32),
            pltpu.VMEM((_CROWS, _E), jnp.float32),
            pltpu.VMEM((_CROWS, _E), jnp.float32),
            pltpu.VMEM((_CROWS, _E), jnp.float32),
            pltpu.SemaphoreType.DMA,
            pltpu.SemaphoreType.DMA,
            pltpu.SemaphoreType.DMA,
            pltpu.SemaphoreType.DMA,
            pltpu.SemaphoreType.DMA,
            pltpu.SemaphoreType.DMA,
        ],
    )
    def _sc_gather(emb_hbm, idx_hbm, out_hbm, idx_v, buf0, buf1, buf2,
                   gs0, gs1, gs2, os0, os1, os2):
        wid = lax.axis_index("s") * _NC + lax.axis_index("c")
        base = wid * _BPW
        bufs = (buf0, buf1, buf2)
        gsems = (gs0, gs1, gs2)
        osems = (os0, os1, os2)

        def pipeline(nch):
            gcp = [None] * nch
            ocp = [None] * nch
            for c in range(min(3, nch)):
                gcp[c] = pltpu.async_copy(
                    emb_hbm.at[idx_v.at[pl.ds(c * _CROWS, _CROWS)]],
                    bufs[c % 3], gsems[c % 3])
            for c in range(nch):
                gcp[c].wait()
                ocp[c] = pltpu.async_copy(
                    bufs[c % 3],
                    out_hbm.at[pl.ds(base + c * _CROWS, _CROWS)],
                    osems[c % 3])
                if c + 3 < nch:
                    ocp[c].wait()
                    gcp[c + 3] = pltpu.async_copy(
                        emb_hbm.at[idx_v.at[pl.ds((c + 3) * _CROWS,
                                                  _CROWS)]],
                        bufs[c % 3], gsems[c % 3])
            for c in range(max(0, nch - 3), nch):
                ocp[c].wait()

        @pl.when(wid < _NW - 1)
        def _full():
            pltpu.sync_copy(idx_hbm.at[pl.ds(base, _BPW)], idx_v)
            pipeline(_CH)

        @pl.when(wid == _NW - 1)
        def _tail():
            pltpu.sync_copy(idx_hbm.at[pl.ds(base, _BPW // 2)],
                            idx_v.at[pl.ds(0, _BPW // 2)])
            pipeline(_CH // 2)

    return _sc_gather


def _dot_t(a, b_t):
    return lax.dot_general(a, b_t, (((1,), (1,)), ((), ())),
                           preferred_element_type=jnp.float32)


def _dot(a, b):
    return lax.dot_general(a, b, (((1,), (0,)), ((), ())),
                           preferred_element_type=jnp.float32)


def _gru_combine(gi, gh, h0):
    r = jax.nn.sigmoid(gi[:, :_E] + gh[:, :_E])
    z = jax.nn.sigmoid(gi[:, _E:2 * _E] + gh[:, _E:2 * _E])
    n = jnp.tanh(gi[:, 2 * _E:] + r * gh[:, 2 * _E:])
    return (1.0 - z) * n + z * h0 if h0 is not None else (1.0 - z) * n


def _tree_body(x_ref, wih_ref, whh_ref, bih_ref, bhh_ref, sw_ref, sb_ref,
               cw_ref, out_ref, h_all):
    bih = bih_ref[:]
    bhh = bhh_ref[:]
    wih_b = wih_ref[:, :]
    whh_b = whh_ref[:, :]
    sw_b = sw_ref[:, :]
    cw_b = cw_ref[:, :]

    def gi_at(prow, m):
        x = x_ref[pl.ds(prow, m), :]
        return _dot_t(x, wih_b) + bih

    def leaf_step(prow, nn):
        m = nn * _BS
        gi = gi_at(prow, m)
        gh = jnp.broadcast_to(bhh, (m, 3 * _E))
        h = _gru_combine(gi, gh, None)
        h_all[pl.ds(prow, m), :] = h
        return h

    def internal_step(prow, crow, nn):
        m = nn * _BS
        hc = h_all[pl.ds(crow, 2 * m), :]
        u = jnp.tanh(_dot(hc, sw_b) + sb_ref[:, :])
        s = jnp.tanh(_dot(u, cw_b))
        s4 = s.reshape(nn, 2, _BS, 1)
        w1 = jax.nn.sigmoid(s4[:, 0] - s4[:, 1])
        hc4 = hc.reshape(nn, 2, _BS, _E)
        h0 = (w1 * hc4[:, 0] + (1.0 - w1) * hc4[:, 1]).reshape(m, _E)
        gi = gi_at(prow, m)
        gh = _dot_t(h0, whh_b) + bhh
        h = _gru_combine(gi, gh, h0)
        h_all[pl.ds(prow, m), :] = h
        return h

    def fold(acc, h, nn):
        m = jnp.max(h.reshape(nn, _BS, _E), axis=0) if nn > 1 else h
        return m if acc is None else jnp.maximum(acc, m)

    acc = None
    for k in range(4):
        acc = fold(acc, leaf_step(3968 + k * 1024, 8), 8)
    for half in range(2):
        acc = fold(acc, internal_step(1920 + half * 1024,
                                      3968 + half * 2048, 8), 8)
    acc = fold(acc, internal_step(896, 1920, 8), 8)
    acc = fold(acc, internal_step(384, 896, 4), 4)
    acc = fold(acc, internal_step(128, 384, 2), 2)
    acc = fold(acc, internal_step(0, 128, 1), 1)
    out_ref[:, :] = acc


def _tree_gru(x_pad, wih, whh, bih, bhh, sw, sb, cw):
    return pl.pallas_call(
        _tree_body,
        in_specs=[pl.BlockSpec(memory_space=pltpu.VMEM)] * 8,
        out_specs=pl.BlockSpec(memory_space=pltpu.VMEM),
        out_shape=jax.ShapeDtypeStruct((_BS, _E), jnp.float32),
        scratch_shapes=[
            pltpu.VMEM((_N * _BS, _E), jnp.float32),
        ],
    )(x_pad, wih, whh, bih, bhh, sw, sb, cw)


def kernel(node_ids, emb, Wih, Whh, bih, bhh, sent_w, sent_b, ctx_w):
    x = _make_sc_gather()(emb, node_ids.reshape(-1))
    return _tree_gru(x, Wih, Whh, bih, bhh, sent_w, sent_b, ctx_w)


# speedup vs baseline: 1.7330x; 1.0221x over previous
import functools

import jax
import jax.numpy as jnp
from jax import lax
from jax.experimental import pallas as pl
from jax.experimental.pallas import tpu as pltpu
from jax.experimental.pallas import tpu_sc as plsc

_E = 512
_BS = 128
_N = 63
_PADN = 64
_ROWS = _PADN * _BS

_NC, _NS = 2, 16
_NW = _NC * _NS
_BPW = _ROWS // _NW
_CH = 4
_CROWS = _BPW // _CH


@functools.cache
def _make_sc_gather():
    mesh = plsc.VectorSubcoreMesh(core_axis_name="c", subcore_axis_name="s")

    @functools.partial(
        pl.kernel,
        mesh=mesh,
        out_type=jax.ShapeDtypeStruct((_N * _BS, _E), jnp.float32),
        scratch_types=[
            pltpu.VMEM((_BPW,), jnp.int32),
            pltpu.VMEM((_CROWS, _E), jnp.float32),
            pltpu.VMEM((_CROWS, _E), jnp.float32),
            pltpu.VMEM((_CROWS, _E), jnp.float32),
            pltpu.SemaphoreType.DMA,
            pltpu.SemaphoreType.DMA,
            pltpu.SemaphoreType.DMA,
            pltpu.SemaphoreType.DMA,
            pltpu.SemaphoreType.DMA,
            pltpu.SemaphoreType.DMA,
        ],
    )
    def _sc_gather(emb_hbm, idx_hbm, out_hbm, idx_v, buf0, buf1, buf2,
                   gs0, gs1, gs2, os0, os1, os2):
        wid = lax.axis_index("s") * _NC + lax.axis_index("c")
        base = wid * _BPW
        bufs = (buf0, buf1, buf2)
        gsems = (gs0, gs1, gs2)
        osems = (os0, os1, os2)

        def pipeline(nch):
            gcp = [None] * nch
            ocp = [None] * nch
            for c in range(min(3, nch)):
                gcp[c] = pltpu.async_copy(
                    emb_hbm.at[idx_v.at[pl.ds(c * _CROWS, _CROWS)]],
                    bufs[c % 3], gsems[c % 3])
            for c in range(nch):
                gcp[c].wait()
                ocp[c] = pltpu.async_copy(
                    bufs[c % 3],
                    out_hbm.at[pl.ds(base + c * _CROWS, _CROWS)],
                    osems[c % 3])
                if c + 3 < nch:
                    ocp[c].wait()
                    gcp[c + 3] = pltpu.async_copy(
                        emb_hbm.at[idx_v.at[pl.ds((c + 3) * _CROWS,
                                                  _CROWS)]],
                        bufs[c % 3], gsems[c % 3])
            for c in range(max(0, nch - 3), nch):
                ocp[c].wait()

        @pl.when(wid < _NW - 1)
        def _full():
            pltpu.sync_copy(idx_hbm.at[pl.ds(base, _BPW)], idx_v)
            pipeline(_CH)

        @pl.when(wid == _NW - 1)
        def _tail():
            pltpu.sync_copy(idx_hbm.at[pl.ds(base, _BPW // 2)],
                            idx_v.at[pl.ds(0, _BPW // 2)])
            pipeline(_CH // 2)

    return _sc_gather


def _dot_t(a, b_t):
    return lax.dot_general(a, b_t, (((1,), (1,)), ((), ())),
                           preferred_element_type=jnp.float32)


def _dot(a, b):
    return lax.dot_general(a, b, (((1,), (0,)), ((), ())),
                           preferred_element_type=jnp.float32)


def _gru_combine(gi, gh, h0):
    r = jax.nn.sigmoid(gi[:, :_E] + gh[:, :_E])
    z = jax.nn.sigmoid(gi[:, _E:2 * _E] + gh[:, _E:2 * _E])
    n = jnp.tanh(gi[:, 2 * _E:] + r * gh[:, 2 * _E:])
    return (1.0 - z) * n + z * h0 if h0 is not None else (1.0 - z) * n


def _tree_body(x_hbm, wih_ref, whh_ref, bih_ref, bhh_ref, sw_ref, sb_ref,
               cw_ref, out_ref, h_all, x_vmem, xsems, isem):
    xcp = []
    for k in range(4):
        r = 3968 + k * 1024
        xcp.append(pltpu.make_async_copy(
            x_hbm.at[pl.ds(r, 1024), :], x_vmem.at[pl.ds(r, 1024), :],
            xsems.at[k]))
        xcp[k].start()
    icp = pltpu.make_async_copy(
        x_hbm.at[pl.ds(0, 3968), :], x_vmem.at[pl.ds(0, 3968), :], isem)
    icp.start()

    bih = bih_ref[:]
    bhh = bhh_ref[:]
    wih_b = wih_ref[:, :]
    whh_b = whh_ref[:, :]
    sw_b = sw_ref[:, :]
    cw_b = cw_ref[:, :]

    def gi_at(prow, m):
        x = x_vmem[pl.ds(prow, m), :]
        return _dot_t(x, wih_b) + bih

    def leaf_step(prow, nn):
        m = nn * _BS
        gi = gi_at(prow, m)
        gh = jnp.broadcast_to(bhh, (m, 3 * _E))
        h = _gru_combine(gi, gh, None)
        h_all[pl.ds(prow, m), :] = h
        return h

    def internal_step(prow, crow, nn):
        m = nn * _BS
        hc = h_all[pl.ds(crow, 2 * m), :]
        u = jnp.tanh(_dot(hc, sw_b) + sb_ref[:, :])
        s = jnp.tanh(_dot(u, cw_b))
        s4 = s.reshape(nn, 2, _BS, 1)
        w1 = jax.nn.sigmoid(s4[:, 0] - s4[:, 1])
        hc4 = hc.reshape(nn, 2, _BS, _E)
        h0 = (w1 * hc4[:, 0] + (1.0 - w1) * hc4[:, 1]).reshape(m, _E)
        gi = gi_at(prow, m)
        gh = _dot_t(h0, whh_b) + bhh
        h = _gru_combine(gi, gh, h0)
        h_all[pl.ds(prow, m), :] = h
        return h

    def fold(acc, h, nn):
        m = jnp.max(h.reshape(nn, _BS, _E), axis=0) if nn > 1 else h
        return m if acc is None else jnp.maximum(acc, m)

    acc = None
    for k in range(4):
        xcp[k].wait()
        acc = fold(acc, leaf_step(3968 + k * 1024, 8), 8)
    icp.wait()
    for half in range(2):
        acc = fold(acc, internal_step(1920 + half * 1024,
                                      3968 + half * 2048, 8), 8)
    acc = fold(acc, internal_step(896, 1920, 8), 8)
    acc = fold(acc, internal_step(384, 896, 4), 4)
    acc = fold(acc, internal_step(128, 384, 2), 2)
    acc = fold(acc, internal_step(0, 128, 1), 1)
    out_ref[:, :] = acc


def _tree_gru(x, wih, whh, bih, bhh, sw, sb, cw):
    return pl.pallas_call(
        _tree_body,
        in_specs=[pl.BlockSpec(memory_space=pltpu.HBM)]
        + [pl.BlockSpec(memory_space=pltpu.VMEM)] * 7,
        out_specs=pl.BlockSpec(memory_space=pltpu.VMEM),
        out_shape=jax.ShapeDtypeStruct((_BS, _E), jnp.float32),
        scratch_shapes=[
            pltpu.VMEM((_N * _BS, _E), jnp.float32),
            pltpu.VMEM((_N * _BS, _E), jnp.float32),
            pltpu.SemaphoreType.DMA((4,)),
            pltpu.SemaphoreType.DMA,
        ],
    )(x, wih, whh, bih, bhh, sw, sb, cw)


def kernel(node_ids, emb, Wih, Whh, bih, bhh, sent_w, sent_b, ctx_w):
    x = _make_sc_gather()(emb, node_ids.reshape(-1))
    return _tree_gru(x, Wih, Whh, bih, bhh, sent_w, sent_b, ctx_w)
